# Initial kernel scaffold; baseline (speedup 1.0000x reference)
import jax
import jax.numpy as jnp
from jax.experimental import pallas as pl


def kernel(hg_adj_matrix__indices, hg_adj_matrix__values, items_emb):
    raise NotImplementedError("write your pallas kernel here")


# speedup vs baseline: 13.5288x; 13.5288x over previous
import functools

import jax
import jax.numpy as jnp
from jax import lax
from jax.experimental import pallas as pl
from jax.experimental.pallas import tpu as pltpu
from jax.experimental.pallas import tpu_sc as plsc

N = 65536
NNZ = 4194304
D = 64
L = 16
NCHUNK = D // L
NTILES = 16
EPT = NNZ // NTILES
SUB = 128
NSUB = 16
BATCH = SUB * NSUB
NBATCH = EPT // BATCH
RPT = N // NTILES
FBLK = 128
NFBLK = RPT // FBLK


@functools.cache
def _build():
  mesh = plsc.VectorSubcoreMesh(core_axis_name="c", subcore_axis_name="s",
                                num_cores=2, num_subcores=NTILES)

  @functools.partial(
      pl.kernel,
      out_type=jax.ShapeDtypeStruct((NCHUNK, N, L), jnp.float32),
      mesh=mesh,
      scratch_types=[
          pltpu.VMEM((NSUB, SUB), jnp.int32),
          pltpu.VMEM((NSUB, SUB), jnp.int32),
          pltpu.VMEM((NSUB, SUB), jnp.float32),
          pltpu.VMEM((NSUB, SUB, L), jnp.float32),
          pltpu.VMEM((FBLK, L), jnp.float32),
          pltpu.VMEM((FBLK, L), jnp.float32),
          pltpu.VMEM((FBLK, L), jnp.float32),
          pltpu.VMEM((FBLK,), jnp.int32),
          pltpu.VMEM_SHARED((N, L), jnp.float32),
          pltpu.SemaphoreType.DMA,
          pltpu.SemaphoreType.DMA,
      ],
      compiler_params=pltpu.CompilerParams(use_tc_tiling_on_sc=False),
  )
  def _hgc_sc(row_hbm, col_hbm, val_hbm, emb_r_hbm, out_hbm,
              row2d, col2d, val2d, gath, zbuf, facc, femb, fidx,
              acc, gsem, ssem):
    cid = lax.axis_index("c")
    tid = lax.axis_index("s")
    iota = lax.broadcasted_iota(jnp.int32, (L,), 0)

    def _zb(m, _):
      zbuf[m, :] = jnp.zeros((L,), jnp.float32)
      return 0
    lax.fori_loop(0, FBLK, _zb, 0)

    for p in range(NCHUNK // 2):
      c = cid * 2 + p

      def _zero(i, _):
        pltpu.sync_copy(zbuf, acc.at[pl.ds(tid * RPT + i * FBLK, FBLK)])
        return 0
      lax.fori_loop(0, NFBLK, _zero, 0)
      plsc.subcore_barrier()

      def _batch(i, _):
        blk = tid * (EPT // SUB) + i * NSUB
        pltpu.sync_copy(row_hbm.at[pl.ds(blk, NSUB)], row2d)
        pltpu.sync_copy(col_hbm.at[pl.ds(blk, NSUB)], col2d)
        pltpu.sync_copy(val_hbm.at[pl.ds(blk, NSUB)], val2d)
        for j in range(NSUB):
          for k in range(SUB // L):
            sl = pl.ds(k * L, L)
            col2d[j, sl] = col2d[j, sl] * NCHUNK + c
        cps = [pltpu.async_copy(emb_r_hbm.at[col2d.at[j]], gath.at[j], gsem)
               for j in range(NSUB)]
        for cp in cps:
          cp.wait()
        for j in range(NSUB):
          def _scale(g, _):
            b0 = g * L
            vv = val2d[j, pl.ds(b0, L)]
            for q in range(L):
              gath[j, b0 + q, :] = gath[j, b0 + q, :] * vv[q]
            return 0
          lax.fori_loop(0, SUB // L, _scale, 0)
        scps = [pltpu.async_copy(gath.at[j], acc.at[row2d.at[j]], ssem,
                                 add=True)
                for j in range(NSUB)]
        for cp in scps:
          cp.wait()
        return 0
      lax.fori_loop(0, NBATCH, _batch, 0)
      plsc.subcore_barrier()

      def _flush(i, _):
        r0 = tid * RPT + i * FBLK
        pltpu.sync_copy(acc.at[pl.ds(r0, FBLK)], facc)
        for k in range(FBLK // L):
          fidx[pl.ds(k * L, L)] = (r0 + k * L + iota) * NCHUNK + c
        pltpu.async_copy(emb_r_hbm.at[fidx], femb, gsem).wait()
        def _comb(m, _):
          facc[m, :] = facc[m, :] * 0.75 + femb[m, :] * 0.25
          return 0
        lax.fori_loop(0, FBLK, _comb, 0)
        pltpu.sync_copy(facc, out_hbm.at[c, pl.ds(r0, FBLK)])
        return 0
      lax.fori_loop(0, NFBLK, _flush, 0)
      plsc.subcore_barrier()

  return _hgc_sc


def kernel(hg_adj_matrix__indices, hg_adj_matrix__values, items_emb):
  row = hg_adj_matrix__indices[0].astype(jnp.int32).reshape(NNZ // SUB, SUB)
  col = hg_adj_matrix__indices[1].astype(jnp.int32).reshape(NNZ // SUB, SUB)
  val = hg_adj_matrix__values.reshape(NNZ // SUB, SUB)
  emb_r = items_emb.reshape(N * NCHUNK, L)
  out_cm = _build()(row, col, val, emb_r)
  return out_cm.transpose(1, 0, 2).reshape(N, D)


# speedup vs baseline: 14.9188x; 1.1027x over previous
import functools

import jax
import jax.numpy as jnp
from jax import lax
from jax.experimental import pallas as pl
from jax.experimental.pallas import tpu as pltpu
from jax.experimental.pallas import tpu_sc as plsc

N = 65536
NNZ = 4194304
D = 64
L = 16
NCHUNK = D // L
NTILES = 16
EPT = NNZ // NTILES
SUB = 128
NSUB = 8
BATCH = SUB * NSUB
NBATCH = EPT // BATCH
RPT = N // NTILES
FBLK = 128
NFBLK = RPT // FBLK


@functools.cache
def _build():
  mesh = plsc.VectorSubcoreMesh(core_axis_name="c", subcore_axis_name="s",
                                num_cores=2, num_subcores=NTILES)

  @functools.partial(
      pl.kernel,
      out_type=jax.ShapeDtypeStruct((NCHUNK, N, L), jnp.float32),
      mesh=mesh,
      scratch_types=[
          pltpu.VMEM((2, NSUB, SUB), jnp.int32),
          pltpu.VMEM((2, NSUB, SUB), jnp.int32),
          pltpu.VMEM((2, NSUB, SUB), jnp.float32),
          pltpu.VMEM((2, BATCH, L), jnp.float32),
          pltpu.VMEM((FBLK, L), jnp.float32),
          pltpu.VMEM((FBLK, L), jnp.float32),
          pltpu.VMEM((FBLK, L), jnp.float32),
          pltpu.VMEM((FBLK,), jnp.int32),
          pltpu.VMEM_SHARED((N, L), jnp.float32),
          pltpu.SemaphoreType.DMA,
          pltpu.SemaphoreType.DMA,
          pltpu.SemaphoreType.DMA,
          pltpu.SemaphoreType.DMA,
      ],
      compiler_params=pltpu.CompilerParams(use_tc_tiling_on_sc=False),
  )
  def _hgc_sc(row_hbm, col_hbm, val_hbm, emb_r_hbm, out_hbm,
              row2d, col2d, val2d, gath, zbuf, facc, femb, fidx,
              acc, gsem0, gsem1, ssem0, ssem1):
    gsems = (gsem0, gsem1)
    ssems = (ssem0, ssem1)
    cid = lax.axis_index("c")
    tid = lax.axis_index("s")
    iota = lax.broadcasted_iota(jnp.int32, (L,), 0)

    def _zb(m, _):
      zbuf[m, :] = jnp.zeros((L,), jnp.float32)
      return 0
    lax.fori_loop(0, FBLK, _zb, 0)

    for p in range(NCHUNK // 2):
      c = cid * 2 + p

      def _zero(i, _):
        pltpu.sync_copy(zbuf, acc.at[pl.ds(tid * RPT + i * FBLK, FBLK)])
        return 0
      lax.fori_loop(0, NFBLK, _zero, 0)
      plsc.subcore_barrier()

      def _prefetch(b, buf):
        blk = tid * (EPT // SUB) + b * NSUB
        pltpu.sync_copy(row_hbm.at[pl.ds(blk, NSUB)], row2d.at[buf])
        pltpu.sync_copy(col_hbm.at[pl.ds(blk, NSUB)], col2d.at[buf])
        pltpu.sync_copy(val_hbm.at[pl.ds(blk, NSUB)], val2d.at[buf])
        for j in range(NSUB):
          for k in range(SUB // L):
            sl = pl.ds(k * L, L)
            col2d[buf, j, sl] = col2d[buf, j, sl] * NCHUNK + c
        for j in range(NSUB):
          pltpu.async_copy(emb_r_hbm.at[col2d.at[buf, j]],
                           gath.at[buf, pl.ds(j * SUB, SUB)], gsems[buf])

      def _drain(sem, buf):
        pltpu.make_async_copy(emb_r_hbm.at[pl.ds(0, BATCH)],
                              gath.at[buf], sem).wait()

      _prefetch(0, 0)
      def _step(i, _):
        for par in range(2):
          b = 2 * i + par
          oth = 1 - par
          _drain(gsems[par], par)
          @pl.when(b > 0)
          def _():
            _drain(ssems[oth], oth)
          @pl.when(b + 1 < NBATCH)
          def _():
            _prefetch(b + 1, oth)
          for j in range(NSUB):
            def _scale(g, _):
              b0 = g * L
              vv = val2d[par, j, pl.ds(b0, L)]
              for q in range(L):
                e = j * SUB + b0 + q
                gath[par, e, :] = gath[par, e, :] * jnp.broadcast_to(
                    vv[q:q + 1], (L,))
              return 0
            lax.fori_loop(0, SUB // L, _scale, 0)
          for j in range(NSUB):
            pltpu.async_copy(gath.at[par, pl.ds(j * SUB, SUB)],
                             acc.at[row2d.at[par, j]], ssems[par], add=True)
        return 0
      lax.fori_loop(0, NBATCH // 2, _step, 0)
      _drain(ssems[1], 1)
      plsc.subcore_barrier()

      def _flush(i, _):
        r0 = tid * RPT + i * FBLK
        pltpu.sync_copy(acc.at[pl.ds(r0, FBLK)], facc)
        for k in range(FBLK // L):
          fidx[pl.ds(k * L, L)] = (r0 + k * L + iota) * NCHUNK + c
        pltpu.async_copy(emb_r_hbm.at[fidx], femb, gsem0).wait()
        def _comb(m, _):
          facc[m, :] = facc[m, :] * 0.75 + femb[m, :] * 0.25
          return 0
        lax.fori_loop(0, FBLK, _comb, 0)
        pltpu.sync_copy(facc, out_hbm.at[c, pl.ds(r0, FBLK)])
        return 0
      lax.fori_loop(0, NFBLK, _flush, 0)
      plsc.subcore_barrier()

  return _hgc_sc


def kernel(hg_adj_matrix__indices, hg_adj_matrix__values, items_emb):
  row = hg_adj_matrix__indices[0].astype(jnp.int32).reshape(NNZ // SUB, SUB)
  col = hg_adj_matrix__indices[1].astype(jnp.int32).reshape(NNZ // SUB, SUB)
  val = hg_adj_matrix__values.reshape(NNZ // SUB, SUB)
  emb_r = items_emb.reshape(N * NCHUNK, L)
  out_cm = _build()(row, col, val, emb_r)
  return out_cm.transpose(1, 0, 2).reshape(N, D)
